# Initial kernel scaffold; baseline (speedup 1.0000x reference)
import jax
import jax.numpy as jnp
from jax.experimental import pallas as pl


def kernel(X, emb, W1, b1, W2, b2, W3, b3):
    raise NotImplementedError("write your pallas kernel here")


# speedup vs baseline: 7.9143x; 7.9143x over previous
import functools

import jax
import jax.numpy as jnp
from jax import lax
from jax.experimental import pallas as pl
from jax.experimental.pallas import tpu as pltpu
from jax.experimental.pallas import tpu_sc as plsc

B, L, EMB = 4096, 200, 128
H1, H2, OUT = 512, 256, 2
NC, NS = 2, 16
NW = NC * NS
BPW = B // NW
LC = 100
NCH = L // LC
NV = EMB // 16


def _make_pool_kernel():
    mesh = plsc.VectorSubcoreMesh(core_axis_name="c", subcore_axis_name="s")

    @functools.partial(
        pl.kernel,
        mesh=mesh,
        out_type=jax.ShapeDtypeStruct((B, EMB), jnp.float32),
        scratch_types=[
            pltpu.VMEM((BPW, NCH, LC), jnp.int32),
            pltpu.VMEM((LC, EMB), jnp.float32),
            pltpu.VMEM((LC, EMB), jnp.float32),
            pltpu.VMEM((BPW, EMB), jnp.float32),
            pltpu.SemaphoreType.DMA,
            pltpu.SemaphoreType.DMA,
        ],
    )
    def pool(emb_hbm, xr_hbm, out_hbm, idx_v, bufa, bufb, out_v, sema, semb):
        wid = lax.axis_index("s") * NC + lax.axis_index("c")
        base = wid * BPW
        pltpu.sync_copy(xr_hbm.at[pl.ds(base, BPW)], idx_v)

        def elem(i, carry):
            ca = pltpu.async_copy(emb_hbm.at[idx_v.at[i, 0]], bufa, sema)
            cb = pltpu.async_copy(emb_hbm.at[idx_v.at[i, 1]], bufb, semb)
            ca.wait()
            cb.wait()

            def red(r, acc):
                return tuple(
                    acc[c]
                    + bufa[r, pl.ds(c * 16, 16)]
                    + bufb[r, pl.ds(c * 16, 16)]
                    for c in range(NV)
                )

            acc0 = tuple(jnp.zeros((16,), jnp.float32) for _ in range(NV))
            acc = lax.fori_loop(0, LC, red, acc0)
            for c in range(NV):
                out_v[i, pl.ds(c * 16, 16)] = acc[c]
            return carry

        lax.fori_loop(0, BPW, elem, 0)
        pltpu.sync_copy(out_v, out_hbm.at[pl.ds(base, BPW)])

    return pool


_pool = _make_pool_kernel()


def _mlp(x, W1, b1, W2, b2, W3, b3):
    BT = 512

    def body(x_ref, w1_ref, b1_ref, w2_ref, b2_ref, w3_ref, b3_ref, o_ref):
        h = x_ref[...] * (1.0 / L)
        h = lax.dot_general(h, w1_ref[...], (((1,), (1,)), ((), ())),
                            preferred_element_type=jnp.float32) + b1_ref[...]
        h = jnp.maximum(h, 0.0)
        h = lax.dot_general(h, w2_ref[...], (((1,), (1,)), ((), ())),
                            preferred_element_type=jnp.float32) + b2_ref[...]
        h = jnp.maximum(h, 0.0)
        h = lax.dot_general(h, w3_ref[...], (((1,), (1,)), ((), ())),
                            preferred_element_type=jnp.float32) + b3_ref[...]
        o_ref[...] = h

    return pl.pallas_call(
        body,
        grid=(B // BT,),
        in_specs=[
            pl.BlockSpec((BT, EMB), lambda i: (i, 0)),
            pl.BlockSpec((H1, EMB), lambda i: (0, 0)),
            pl.BlockSpec((1, H1), lambda i: (0, 0)),
            pl.BlockSpec((H2, H1), lambda i: (0, 0)),
            pl.BlockSpec((1, H2), lambda i: (0, 0)),
            pl.BlockSpec((OUT, H2), lambda i: (0, 0)),
            pl.BlockSpec((1, OUT), lambda i: (0, 0)),
        ],
        out_specs=pl.BlockSpec((BT, OUT), lambda i: (i, 0)),
        out_shape=jax.ShapeDtypeStruct((B, OUT), jnp.float32),
    )(x, W1, b1, W2, b2, W3, b3)


def kernel(X, emb, W1, b1, W2, b2, W3, b3):
    xr = X.astype(jnp.int32).reshape(B, NCH, LC)
    pooled = _pool(emb, xr)
    return _mlp(pooled, W1, b1.reshape(1, H1), W2, b2.reshape(1, H2),
                W3, b3.reshape(1, OUT))


# speedup vs baseline: 13.0805x; 1.6528x over previous
import functools

import jax
import jax.numpy as jnp
from jax import lax
from jax.experimental import pallas as pl
from jax.experimental.pallas import tpu as pltpu
from jax.experimental.pallas import tpu_sc as plsc

B, L, EMB = 4096, 200, 128
H1, H2, OUT = 512, 256, 2
NC, NS = 2, 16
NW = NC * NS
BPW = B // NW
LC = 100
NCH = L // LC
NV = EMB // 16


def _make_pool_kernel():
    mesh = plsc.VectorSubcoreMesh(core_axis_name="c", subcore_axis_name="s")

    @functools.partial(
        pl.kernel,
        mesh=mesh,
        out_type=jax.ShapeDtypeStruct((B, EMB), jnp.float32),
        scratch_types=[
            pltpu.VMEM((BPW, NCH, LC), jnp.int32),
            pltpu.VMEM((LC, EMB), jnp.float32),
            pltpu.VMEM((LC, EMB), jnp.float32),
            pltpu.VMEM((LC, EMB), jnp.float32),
            pltpu.VMEM((LC, EMB), jnp.float32),
            pltpu.VMEM((BPW, EMB), jnp.float32),
            pltpu.SemaphoreType.DMA,
            pltpu.SemaphoreType.DMA,
            pltpu.SemaphoreType.DMA,
            pltpu.SemaphoreType.DMA,
        ],
    )
    def pool(emb_hbm, xr_hbm, out_hbm, idx_v,
             bufa0, bufb0, bufa1, bufb1, out_v, sa0, sb0, sa1, sb1):
        wid = lax.axis_index("s") * NC + lax.axis_index("c")
        base = wid * BPW
        pltpu.sync_copy(xr_hbm.at[pl.ds(base, BPW)], idx_v)

        bufs = ((bufa0, bufb0, sa0, sb0), (bufa1, bufb1, sa1, sb1))

        def start(i, s):
            bufa, bufb, sema, semb = bufs[s]
            pltpu.async_copy(emb_hbm.at[idx_v.at[i, 0]], bufa, sema)
            pltpu.async_copy(emb_hbm.at[idx_v.at[i, 1]], bufb, semb)

        def finish(i, s):
            bufa, bufb, sema, semb = bufs[s]
            pltpu.make_async_copy(emb_hbm.at[idx_v.at[i, 0]], bufa, sema).wait()
            pltpu.make_async_copy(emb_hbm.at[idx_v.at[i, 1]], bufb, semb).wait()

            def red(r, acc):
                return tuple(
                    acc[c]
                    + bufa[r, pl.ds(c * 16, 16)]
                    + bufb[r, pl.ds(c * 16, 16)]
                    for c in range(NV)
                )

            acc0 = tuple(jnp.zeros((16,), jnp.float32) for _ in range(NV))
            acc = lax.fori_loop(0, LC, red, acc0)
            for c in range(NV):
                out_v[i, pl.ds(c * 16, 16)] = acc[c]

        start(0, 0)

        def pair(p, carry):
            i = 2 * p
            start(i + 1, 1)
            finish(i, 0)

            @pl.when(p < BPW // 2 - 1)
            def _():
                start(i + 2, 0)

            finish(i + 1, 1)
            return carry

        lax.fori_loop(0, BPW // 2, pair, 0)
        pltpu.sync_copy(out_v, out_hbm.at[pl.ds(base, BPW)])

    return pool


_pool = _make_pool_kernel()


def _mlp(x, W1, b1, W2, b2, W3, b3):
    BT = 512

    def body(x_ref, w1_ref, b1_ref, w2_ref, b2_ref, w3_ref, b3_ref, o_ref):
        h = x_ref[...] * (1.0 / L)
        h = lax.dot_general(h, w1_ref[...], (((1,), (1,)), ((), ())),
                            preferred_element_type=jnp.float32) + b1_ref[...]
        h = jnp.maximum(h, 0.0)
        h = lax.dot_general(h, w2_ref[...], (((1,), (1,)), ((), ())),
                            preferred_element_type=jnp.float32) + b2_ref[...]
        h = jnp.maximum(h, 0.0)
        h = lax.dot_general(h, w3_ref[...], (((1,), (1,)), ((), ())),
                            preferred_element_type=jnp.float32) + b3_ref[...]
        o_ref[...] = h

    return pl.pallas_call(
        body,
        grid=(B // BT,),
        in_specs=[
            pl.BlockSpec((BT, EMB), lambda i: (i, 0)),
            pl.BlockSpec((H1, EMB), lambda i: (0, 0)),
            pl.BlockSpec((1, H1), lambda i: (0, 0)),
            pl.BlockSpec((H2, H1), lambda i: (0, 0)),
            pl.BlockSpec((1, H2), lambda i: (0, 0)),
            pl.BlockSpec((OUT, H2), lambda i: (0, 0)),
            pl.BlockSpec((1, OUT), lambda i: (0, 0)),
        ],
        out_specs=pl.BlockSpec((BT, OUT), lambda i: (i, 0)),
        out_shape=jax.ShapeDtypeStruct((B, OUT), jnp.float32),
    )(x, W1, b1, W2, b2, W3, b3)


def kernel(X, emb, W1, b1, W2, b2, W3, b3):
    xr = X.astype(jnp.int32).reshape(B, NCH, LC)
    pooled = _pool(emb, xr)
    return _mlp(pooled, W1, b1.reshape(1, H1), W2, b2.reshape(1, H2),
                W3, b3.reshape(1, OUT))
